# baseline (speedup 1.0000x reference)
import functools

import jax
import jax.numpy as jnp
from jax import lax
from jax.experimental import pallas as pl
from jax.experimental.pallas import tpu as pltpu
from jax.experimental.pallas import tpu_sc as plsc

NC = 2
NS = 16
L = 16
K = 128

F32 = jnp.float32
I32 = jnp.int32


def _sc_mesh():
  return plsc.VectorSubcoreMesh(
      core_axis_name="c", subcore_axis_name="s", num_cores=NC, num_subcores=NS)


def _cvec(c):
  return jnp.broadcast_to(c, (L,)).astype(I32)


NBUF = 2


def _csz(rpt, ke):
  c = (min(rpt, ke, 128) // 16) * 16
  while rpt % c:
    c -= 16
  return c


def _make_edge_kernel(EPAD, NROW, TROW, NBUF=NBUF, KE=K):
  CH = EPAD // (NS * KE)
  assert CH % NBUF == 0
  RPT = NROW // NS
  CSZ = _csz(RPT, KE)
  NCO = RPT // CSZ

  @functools.partial(
      pl.kernel,
      out_type=jax.ShapeDtypeStruct((2 * NROW, 128), F32),
      mesh=_sc_mesh(),
      scratch_types=[
          pltpu.VMEM_SHARED((NROW, 128), F32),
          [pltpu.VMEM((3 * KE,), I32)] * NBUF,
          [pltpu.VMEM((KE,), I32)] * NBUF,
          [pltpu.VMEM((KE,), I32)] * NBUF,
          [pltpu.VMEM((KE,), I32)] * NBUF,
          [pltpu.VMEM((KE, 128), F32)] * NBUF,
          pltpu.VMEM((KE, 128), F32),
          pltpu.VMEM((CSZ,), I32),
          [pltpu.SemaphoreType.DMA] * NBUF,
          [pltpu.SemaphoreType.DMA] * NBUF,
          [pltpu.SemaphoreType.DMA] * NBUF,
          [pltpu.SemaphoreType.DMA] * NBUF,
      ],
  )
  def edge_k(a_hbm, bn_hbm, idx_hbm, row2_hbm, out_hbm,
             acc, idxb, gsrc, gdst, sidx, abuf, bbuf, oidx,
             sema, semb, sems, semi):
    c = lax.axis_index("c")
    s = lax.axis_index("s")
    cv = _cvec(c)

    def zero_row(e, carry):
      for g in range(8):
        abuf[0][e, pl.ds(g * L, L)] = jnp.zeros((L,), F32)
      return carry
    lax.fori_loop(0, KE, zero_row, 0)
    for q in range(RPT // CSZ):
      pltpu.sync_copy(abuf[0].at[pl.ds(0, CSZ)],
                      acc.at[pl.ds(s * RPT + q * CSZ, CSZ)])
    plsc.subcore_barrier()

    def group(jj, carry):
      for b in range(NBUF):
        base = ((s * CH + jj * NBUF + b) * 3) * KE
        pltpu.async_copy(idx_hbm.at[pl.ds(base, 3 * KE)], idxb[b], semi[b])
      for b in range(NBUF):
        base = ((s * CH + jj * NBUF + b) * 3) * KE
        pltpu.make_async_copy(idx_hbm.at[pl.ds(base, 3 * KE)], idxb[b],
                              semi[b]).wait()
        for g in range(KE // L):
          gsrc[b][pl.ds(g * L, L)] = idxb[b][pl.ds(g * L, L)] + cv
          gdst[b][pl.ds(g * L, L)] = idxb[b][pl.ds(KE + g * L, L)] + cv
          sidx[b][pl.ds(g * L, L)] = idxb[b][pl.ds(2 * KE + g * L, L)]
        pltpu.async_copy(a_hbm.at[gsrc[b]], abuf[b], sema[b])
        if b == 0:
          pltpu.async_copy(bn_hbm.at[gdst[0]], bbuf, semb[0])
      for b in range(NBUF):
        pltpu.make_async_copy(a_hbm.at[gsrc[b]], abuf[b], sema[b]).wait()
        pltpu.make_async_copy(bn_hbm.at[gdst[b]], bbuf, semb[b]).wait()
        a_ = abuf[b]

        def msg_row(e, inner):
          for g in range(8):
            va = a_[e, pl.ds(g * L, L)]
            vb = bbuf[e, pl.ds(g * L, L)]
            a_[e, pl.ds(g * L, L)] = jnp.maximum(va + vb, 0.0)
          return inner
        lax.fori_loop(0, KE, msg_row, 0)
        if b + 1 < NBUF:
          pltpu.async_copy(bn_hbm.at[gdst[b + 1]], bbuf, semb[b + 1])
        pltpu.async_copy(abuf[b], acc.at[sidx[b]], sems[b], add=True)
      for b in range(NBUF):
        pltpu.make_async_copy(abuf[b], acc.at[sidx[b]], sems[b]).wait()
      return carry
    lax.fori_loop(0, CH // NBUF, group, 0)
    plsc.subcore_barrier()

    for q in range(NCO):
      r0 = s * RPT + q * CSZ
      pltpu.sync_copy(acc.at[pl.ds(r0, CSZ)], abuf[0].at[pl.ds(0, CSZ)])
      pltpu.sync_copy(row2_hbm.at[pl.ds(r0, CSZ)], oidx)
      for g in range(CSZ // L):
        oidx[pl.ds(g * L, L)] = oidx[pl.ds(g * L, L)] + cv
      pltpu.sync_copy(abuf[0].at[pl.ds(0, CSZ)], out_hbm.at[oidx])

  return edge_k


def _make_pool_kernel(NP_, NROW):
  CH = NP_ // (NS * K)
  RPT = NROW // NS
  CSZ = min(128, RPT)
  NCO = RPT // CSZ

  @functools.partial(
      pl.kernel,
      out_type=jax.ShapeDtypeStruct((2 * NROW, 128), F32),
      mesh=_sc_mesh(),
      scratch_types=[
          pltpu.VMEM_SHARED((NROW, 128), F32),
          pltpu.VMEM((K,), I32),
          pltpu.VMEM((K,), I32),
          pltpu.VMEM((K,), I32),
          pltpu.VMEM((K, 128), F32),
          pltpu.VMEM((CSZ,), I32),
          pltpu.SemaphoreType.DMA,
      ],
  )
  def pool_k(x_hbm, labs_hbm, row2_hbm, sums_hbm,
             accs, rraw, gidx, sidx, xbuf, oidx, sem):
    c = lax.axis_index("c")
    s = lax.axis_index("s")
    cv = _cvec(c)

    def fill_row(e, carry):
      for g in range(8):
        xbuf[e, pl.ds(g * L, L)] = jnp.zeros((L,), F32)
      return carry
    lax.fori_loop(0, K, fill_row, 0)
    for q in range(RPT // CSZ):
      pltpu.sync_copy(xbuf.at[pl.ds(0, CSZ)],
                      accs.at[pl.ds(s * RPT + q * CSZ, CSZ)])
    plsc.subcore_barrier()

    def chunk(j, carry):
      base = (s * CH + j) * K
      pltpu.sync_copy(row2_hbm.at[pl.ds(base, K)], rraw)
      pltpu.sync_copy(labs_hbm.at[pl.ds(base, K)], sidx)
      for g in range(K // L):
        gidx[pl.ds(g * L, L)] = rraw[pl.ds(g * L, L)] + cv
      pltpu.async_copy(x_hbm.at[gidx], xbuf, sem).wait()
      pltpu.sync_copy(xbuf, accs.at[sidx], add=True)
      return carry
    lax.fori_loop(0, CH, chunk, 0)
    plsc.subcore_barrier()

    for q in range(NCO):
      r0 = s * RPT + q * CSZ
      pltpu.sync_copy(accs.at[pl.ds(r0, CSZ)], xbuf.at[pl.ds(0, CSZ)])
      pltpu.sync_copy(row2_hbm.at[pl.ds(r0, CSZ)], oidx)
      for g in range(CSZ // L):
        oidx[pl.ds(g * L, L)] = oidx[pl.ds(g * L, L)] + cv
      pltpu.sync_copy(xbuf.at[pl.ds(0, CSZ)], sums_hbm.at[oidx])

  return pool_k


def _make_gather1_kernel(NP_, TROW, KK=128):
  CH = NP_ // (NS * KK)

  @functools.partial(
      pl.kernel,
      out_type=jax.ShapeDtypeStruct((2 * NP_, 128), F32),
      mesh=_sc_mesh(),
      scratch_types=[
          pltpu.VMEM((KK,), I32),
          pltpu.VMEM((KK,), I32),
          pltpu.VMEM((KK,), I32),
          pltpu.VMEM((KK, 128), F32),
          pltpu.SemaphoreType.DMA,
      ],
  )
  def g1_k(t_hbm, lab2_hbm, row2_hbm, out_hbm, rraw, gidx, oidx, buf, sem):
    c = lax.axis_index("c")
    s = lax.axis_index("s")
    cv = _cvec(c)

    def chunk(j, carry):
      base = (s * CH + j) * KK
      pltpu.sync_copy(lab2_hbm.at[pl.ds(base, KK)], rraw)
      pltpu.sync_copy(row2_hbm.at[pl.ds(base, KK)], oidx)
      for g in range(KK // L):
        gidx[pl.ds(g * L, L)] = rraw[pl.ds(g * L, L)] + cv
        oidx[pl.ds(g * L, L)] = oidx[pl.ds(g * L, L)] + cv
      pltpu.async_copy(t_hbm.at[gidx], buf, sem).wait()
      pltpu.sync_copy(buf, out_hbm.at[oidx])
      return carry
    lax.fori_loop(0, CH, chunk, 0)

  return g1_k


def _make_unpool2_kernel(NP_, TROW, KK=128):
  CH = NP_ // (NS * KK)

  @functools.partial(
      pl.kernel,
      out_type=(jax.ShapeDtypeStruct((2 * NP_, 128), F32),
                jax.ShapeDtypeStruct((2 * NP_, 128), F32)),
      mesh=_sc_mesh(),
      scratch_types=[
          pltpu.VMEM((KK,), I32),
          pltpu.VMEM((KK,), I32),
          pltpu.VMEM((KK,), I32),
          pltpu.VMEM((KK, 128), F32),
          pltpu.SemaphoreType.DMA,
      ],
  )
  def up_k(t5_hbm, u6_hbm, p6_hbm, lab2_hbm, row2_hbm, h5_hbm, a6_hbm,
           rraw, gidx, oidx, buf, sem):
    c = lax.axis_index("c")
    s = lax.axis_index("s")
    cv = _cvec(c)

    def chunk(j, carry):
      base = (s * CH + j) * KK
      pltpu.sync_copy(lab2_hbm.at[pl.ds(base, KK)], rraw)
      pltpu.sync_copy(row2_hbm.at[pl.ds(base, KK)], oidx)
      for g in range(KK // L):
        gidx[pl.ds(g * L, L)] = rraw[pl.ds(g * L, L)] + cv
        oidx[pl.ds(g * L, L)] = oidx[pl.ds(g * L, L)] + cv
      pltpu.async_copy(t5_hbm.at[gidx], buf, sem).wait()
      pltpu.sync_copy(buf, h5_hbm.at[oidx])
      pltpu.async_copy(u6_hbm.at[gidx], buf, sem).wait()
      pltpu.async_copy(p6_hbm.at[oidx], buf, sem, add=True).wait()
      pltpu.sync_copy(buf, a6_hbm.at[oidx])
      return carry
    lax.fori_loop(0, CH, chunk, 0)

  return up_k


def _rows_spec(br, d):
  return pl.BlockSpec((br, d), lambda i: (i, 0))


def _full_spec(shape):
  return pl.BlockSpec(shape, lambda i: tuple(0 for _ in shape))


def _tc1_body(feat, pts, g1, w1a, w1b, b1, wm2a, wm2b, bm2, wm6b,
              h1_o, a2_o, b2n_o, p6_o, b6n_o):
  x1 = (jnp.dot(feat[...], w1a[...], preferred_element_type=F32)
        + jnp.dot(pts[...], w1b[...], preferred_element_type=F32) + b1[...])
  h1 = jnp.maximum(x1 - g1[...], 0.0)
  p2 = jnp.dot(pts[...], wm2b[...], preferred_element_type=F32)
  a2 = jnp.dot(h1, wm2a[...], preferred_element_type=F32) + p2 + bm2[...]
  p6 = jnp.dot(pts[...], wm6b[...], preferred_element_type=F32)
  h1_o[...] = h1
  a2_o[...] = a2
  b2n_o[...] = -p2
  p6_o[...] = p6
  b6n_o[...] = -p6


def _tc2_body(agg, wu, bu, hres, out):
  out[...] = jnp.maximum(
      jnp.dot(agg[...], wu[...], preferred_element_type=F32) + bu[...], 0.0
  ) + hres[...]


def _tc3_body(sums, cnt, cc, w3, b3, wm4a, wm4b, bm4,
              h3_o, a4_o, b4n_o):
  inv = 1.0 / jnp.maximum(cnt[...][:, 0:1], 1.0)
  pooled = sums[...] * inv
  h3 = jnp.maximum(jnp.dot(pooled, w3[...], preferred_element_type=F32)
                   + b3[...], 0.0)
  q4 = jnp.dot(cc[...], wm4b[...], preferred_element_type=F32)
  a4 = jnp.dot(h3, wm4a[...], preferred_element_type=F32) + q4 + bm4[...]
  h3_o[...] = h3
  a4_o[...] = a4
  b4n_o[...] = -q4


def _tc4_body(agg4, wu4, bu4, h3, w5, b5, wm6a, bm6, t5_o, u6_o):
  h4 = jnp.maximum(jnp.dot(agg4[...], wu4[...], preferred_element_type=F32)
                   + bu4[...], 0.0) + h3[...]
  t5 = jnp.maximum(jnp.dot(h4, w5[...], preferred_element_type=F32)
                   + b5[...], 0.0)
  u6 = jnp.dot(t5, wm6a[...], preferred_element_type=F32) + bm6[...]
  t5_o[...] = t5
  u6_o[...] = u6


def _tc5_body(agg6, wu6, bu6, h5, h2, wc, bc, out):
  h6 = jnp.maximum(jnp.dot(agg6[...], wu6[...], preferred_element_type=F32)
                   + bu6[...], 0.0) + h5[...]
  out[...] = jnp.dot(h6 + h2[...], wc[...],
                     preferred_element_type=F32) + bc[...]


def _tc0_body(cc, w1b, q1_o):
  q1_o[...] = jnp.dot(cc[...], w1b[...], preferred_element_type=F32)


def _hist_body(nbins, lab_ref, out_ref):
  i = pl.program_id(0)

  @pl.when(i == 0)
  def _():
    out_ref[...] = jnp.zeros_like(out_ref)

  lab = lab_ref[0, 0, :]
  nl = lab.shape[0]
  bins = lax.broadcasted_iota(I32, (nbins, nl), 0)
  oh = (bins == lab[None, :]).astype(F32)
  cnt = jnp.sum(oh, axis=1)
  out_ref[...] += jnp.broadcast_to(cnt[:, None], (nbins, 128))


def kernel(features, points, cluster_centers, l0_edges, l1_edges, labels,
           W1, b1, Wm2, bm2, Wu2, bu2, W3, b3, Wm4, bm4, Wu4, bu4,
           W5, b5, Wm6, bm6, Wu6, bu6, Wc, bc):
  n, d = features.shape
  m = cluster_centers.shape[0]
  e0 = l0_edges.shape[1]
  e1 = l1_edges.shape[1]
  c_out = Wc.shape[1]

  npad = ((n + NS * K - 1) // (NS * K)) * (NS * K)
  mpad = ((m + NS * 64 - 1) // (NS * 64)) * (NS * 64)
  KE0, KE1 = 112, 128
  ek0 = NS * KE0 * NBUF
  ek1 = NS * KE1 * NBUF
  e0pad = ((e0 + ek0 - 1) // ek0) * ek0
  e1pad = ((e1 + ek1 - 1) // ek1) * ek1
  DUMN = n + 2
  DUMM = m + 2

  f32 = functools.partial(jnp.asarray, dtype=F32)
  featp = jnp.pad(f32(features), ((0, npad - n), (0, 0)))
  ptsp = jnp.pad(f32(points), ((0, npad - n), (0, 0)))
  ccp = jnp.pad(f32(cluster_centers), ((0, mpad - m), (0, 0)))

  lab = labels.astype(I32)
  lab2 = jnp.pad(2 * lab, (0, npad - n))
  labS = jnp.pad(lab, (0, npad - n), constant_values=DUMM)
  row2n = (2 * jnp.arange(npad, dtype=I32))
  row2m = (2 * jnp.arange(mpad, dtype=I32))

  def pack_idx(edges, ne, epad, dum, kk):
    src = edges[0].astype(I32)
    dst = edges[1].astype(I32)
    src2 = jnp.pad(2 * src, (0, epad - ne))
    dst2 = jnp.pad(2 * dst, (0, epad - ne))
    dsts = jnp.pad(dst, (0, epad - ne), constant_values=dum)
    return jnp.stack([src2.reshape(-1, kk), dst2.reshape(-1, kk),
                      dsts.reshape(-1, kk)], axis=1).reshape(-1)

  idx0 = pack_idx(l0_edges, e0, e0pad, DUMN, KE0)
  idx1 = pack_idx(l1_edges, e1, e1pad, DUMM, KE1)

  r1 = lambda v: f32(v).reshape(1, -1)
  W1a, W1b = f32(W1[:d]), f32(W1[d:])
  Wm2a, Wm2b = f32(Wm2[:d]), f32(Wm2[d:])
  Wm4a, Wm4b = f32(Wm4[:d]), f32(Wm4[d:])
  Wm6a, Wm6b = f32(Wm6[:d]), f32(Wm6[d:])

  BRN = 1024
  GN = npad // BRN

  q1 = pl.pallas_call(
      _tc0_body,
      grid=(1,),
      in_specs=[_rows_spec(mpad, 3), _full_spec((3, d))],
      out_specs=_rows_spec(mpad, d),
      out_shape=jax.ShapeDtypeStruct((mpad, d), F32),
  )(ccp, W1b)

  g1_2n = _make_gather1_kernel(npad, 2 * mpad)(
      q1.reshape(2 * mpad, 128), lab2, row2n)
  g1 = g1_2n.reshape(npad, d)

  h1, a2, b2n, p6, b6n = pl.pallas_call(
      _tc1_body,
      grid=(GN,),
      in_specs=[_rows_spec(BRN, d), _rows_spec(BRN, 3), _rows_spec(BRN, d),
                _full_spec((d, d)), _full_spec((3, d)), _full_spec((1, d)),
                _full_spec((d, d)), _full_spec((3, d)), _full_spec((1, d)),
                _full_spec((3, d))],
      out_specs=[_rows_spec(BRN, d)] * 5,
      out_shape=[jax.ShapeDtypeStruct((npad, d), F32)] * 5,
  )(featp, ptsp, g1, W1a, W1b, r1(b1), Wm2a, Wm2b, r1(bm2), Wm6b)

  edge_n = _make_edge_kernel(e0pad, npad, 2 * npad, KE=KE0)

  agg2_2n = edge_n(a2.reshape(2 * npad, 128), b2n.reshape(2 * npad, 128),
                   idx0, row2n)
  agg2 = agg2_2n.reshape(npad, d)

  h2 = pl.pallas_call(
      _tc2_body,
      grid=(GN,),
      in_specs=[_rows_spec(BRN, d), _full_spec((d, d)), _full_spec((1, d)),
                _rows_spec(BRN, d)],
      out_specs=_rows_spec(BRN, d),
      out_shape=jax.ShapeDtypeStruct((npad, d), F32),
  )(agg2, f32(Wu2), r1(bu2), h1)

  sums_2n = _make_pool_kernel(npad, mpad)(
      h2.reshape(2 * npad, 128), labS, row2n)
  sums = sums_2n.reshape(mpad, d)

  LBLK = 1024
  cnt = pl.pallas_call(
      functools.partial(_hist_body, mpad),
      grid=(npad // LBLK,),
      in_specs=[pl.BlockSpec((1, 1, LBLK), lambda i: (i, 0, 0))],
      out_specs=_full_spec((mpad, 128)),
      out_shape=jax.ShapeDtypeStruct((mpad, 128), F32),
  )(labS.reshape(npad // LBLK, 1, LBLK))

  h3, a4, b4n = pl.pallas_call(
      _tc3_body,
      grid=(1,),
      in_specs=[_rows_spec(mpad, d), _rows_spec(mpad, 128), _rows_spec(mpad, 3),
                _full_spec((d, d)), _full_spec((1, d)),
                _full_spec((d, d)), _full_spec((3, d)), _full_spec((1, d))],
      out_specs=[_rows_spec(mpad, d)] * 3,
      out_shape=[jax.ShapeDtypeStruct((mpad, d), F32)] * 3,
  )(sums, cnt, ccp, f32(W3), r1(b3), Wm4a, Wm4b, r1(bm4))

  agg4_2n = _make_edge_kernel(e1pad, mpad, 2 * mpad, KE=KE1)(
      a4.reshape(2 * mpad, 128), b4n.reshape(2 * mpad, 128),
      idx1, row2m)
  agg4 = agg4_2n.reshape(mpad, d)

  t5, u6 = pl.pallas_call(
      _tc4_body,
      grid=(1,),
      in_specs=[_rows_spec(mpad, d), _full_spec((d, d)), _full_spec((1, d)),
                _rows_spec(mpad, d), _full_spec((d, d)), _full_spec((1, d)),
                _full_spec((d, d)), _full_spec((1, d))],
      out_specs=[_rows_spec(mpad, d)] * 2,
      out_shape=[jax.ShapeDtypeStruct((mpad, d), F32)] * 2,
  )(agg4, f32(Wu4), r1(bu4), h3, f32(W5), r1(b5), Wm6a, r1(bm6))

  h5_2n, a6_2n = _make_unpool2_kernel(npad, 2 * mpad)(
      t5.reshape(2 * mpad, 128), u6.reshape(2 * mpad, 128),
      p6.reshape(2 * npad, 128), lab2, row2n)
  h5 = h5_2n.reshape(npad, d)

  agg6_2n = edge_n(a6_2n, b6n.reshape(2 * npad, 128),
                   idx0, row2n)
  agg6 = agg6_2n.reshape(npad, d)

  out = pl.pallas_call(
      _tc5_body,
      grid=(GN,),
      in_specs=[_rows_spec(BRN, d), _full_spec((d, d)), _full_spec((1, d)),
                _rows_spec(BRN, d), _rows_spec(BRN, d),
                _full_spec((d, c_out)), _full_spec((1, c_out))],
      out_specs=_rows_spec(BRN, c_out),
      out_shape=jax.ShapeDtypeStruct((npad, c_out), F32),
  )(agg6, f32(Wu6), r1(bu6), h5, h2, f32(Wc), r1(bc))

  return out[:n]


# speedup vs baseline: 1.0719x; 1.0719x over previous
import functools

import jax
import jax.numpy as jnp
from jax import lax
from jax.experimental import pallas as pl
from jax.experimental.pallas import tpu as pltpu
from jax.experimental.pallas import tpu_sc as plsc

NC = 2
NS = 16
L = 16
K = 128

F32 = jnp.float32
I32 = jnp.int32


def _sc_mesh():
  return plsc.VectorSubcoreMesh(
      core_axis_name="c", subcore_axis_name="s", num_cores=NC, num_subcores=NS)


def _cvec(c):
  return jnp.broadcast_to(c, (L,)).astype(I32)


NBUF = 2


def _csz(rpt, ke):
  c = (min(rpt, ke, 128) // 16) * 16
  while rpt % c:
    c -= 16
  return c


def _make_edge_kernel(EPAD, NROW, TROW, NBUF=NBUF, KE=K):
  CH = EPAD // (NS * KE)
  assert CH % NBUF == 0
  RPT = NROW // NS
  CSZ = _csz(RPT, KE)
  NCO = RPT // CSZ

  @functools.partial(
      pl.kernel,
      out_type=jax.ShapeDtypeStruct((2 * NROW, 128), F32),
      mesh=_sc_mesh(),
      scratch_types=[
          pltpu.VMEM_SHARED((NROW, 128), F32),
          [pltpu.VMEM((3 * KE,), I32)] * NBUF,
          [pltpu.VMEM((KE,), I32)] * NBUF,
          [pltpu.VMEM((KE,), I32)] * NBUF,
          [pltpu.VMEM((KE,), I32)] * NBUF,
          [pltpu.VMEM((KE, 128), F32)] * NBUF,
          pltpu.VMEM((KE, 128), F32),
          pltpu.VMEM((CSZ,), I32),
          [pltpu.SemaphoreType.DMA] * NBUF,
          [pltpu.SemaphoreType.DMA] * NBUF,
          [pltpu.SemaphoreType.DMA] * NBUF,
          [pltpu.SemaphoreType.DMA] * NBUF,
      ],
  )
  def edge_k(a_hbm, bn_hbm, idx_hbm, row2_hbm, out_hbm,
             acc, idxb, gsrc, gdst, sidx, abuf, bbuf, oidx,
             sema, semb, sems, semi):
    c = lax.axis_index("c")
    s = lax.axis_index("s")
    cv = _cvec(c)

    def zero_row(e, carry):
      for g in range(8):
        abuf[0][e, pl.ds(g * L, L)] = jnp.zeros((L,), F32)
      return carry
    lax.fori_loop(0, KE, zero_row, 0)
    for q in range(RPT // CSZ):
      pltpu.sync_copy(abuf[0].at[pl.ds(0, CSZ)],
                      acc.at[pl.ds(s * RPT + q * CSZ, CSZ)])
    plsc.subcore_barrier()

    def group(jj, carry):
      for b in range(NBUF):
        base = ((s * CH + jj * NBUF + b) * 3) * KE
        pltpu.sync_copy(idx_hbm.at[pl.ds(base, 3 * KE)], idxb[b])
        for g in range(KE // L):
          gsrc[b][pl.ds(g * L, L)] = idxb[b][pl.ds(g * L, L)] + cv
          gdst[b][pl.ds(g * L, L)] = idxb[b][pl.ds(KE + g * L, L)] + cv
          sidx[b][pl.ds(g * L, L)] = idxb[b][pl.ds(2 * KE + g * L, L)]
        pltpu.async_copy(a_hbm.at[gsrc[b]], abuf[b], sema[b])
        if b == 0:
          pltpu.async_copy(bn_hbm.at[gdst[0]], bbuf, semb[0])
      for b in range(NBUF):
        pltpu.make_async_copy(a_hbm.at[gsrc[b]], abuf[b], sema[b]).wait()
        pltpu.make_async_copy(bn_hbm.at[gdst[b]], bbuf, semb[b]).wait()
        a_ = abuf[b]

        def msg_row(e, inner):
          for g in range(8):
            va = a_[e, pl.ds(g * L, L)]
            vb = bbuf[e, pl.ds(g * L, L)]
            a_[e, pl.ds(g * L, L)] = jnp.maximum(va + vb, 0.0)
          return inner
        lax.fori_loop(0, KE, msg_row, 0)
        if b + 1 < NBUF:
          pltpu.async_copy(bn_hbm.at[gdst[b + 1]], bbuf, semb[b + 1])
        pltpu.async_copy(abuf[b], acc.at[sidx[b]], sems[b], add=True)
      for b in range(NBUF):
        pltpu.make_async_copy(abuf[b], acc.at[sidx[b]], sems[b]).wait()
      return carry
    lax.fori_loop(0, CH // NBUF, group, 0)
    plsc.subcore_barrier()

    for q in range(NCO):
      r0 = s * RPT + q * CSZ
      pltpu.sync_copy(acc.at[pl.ds(r0, CSZ)], abuf[0].at[pl.ds(0, CSZ)])
      pltpu.sync_copy(row2_hbm.at[pl.ds(r0, CSZ)], oidx)
      for g in range(CSZ // L):
        oidx[pl.ds(g * L, L)] = oidx[pl.ds(g * L, L)] + cv
      pltpu.sync_copy(abuf[0].at[pl.ds(0, CSZ)], out_hbm.at[oidx])

  return edge_k


def _make_pool_kernel(NP_, NROW):
  CH = NP_ // (NS * K)
  RPT = NROW // NS
  CSZ = min(128, RPT)
  NCO = RPT // CSZ

  @functools.partial(
      pl.kernel,
      out_type=jax.ShapeDtypeStruct((2 * NROW, 128), F32),
      mesh=_sc_mesh(),
      scratch_types=[
          pltpu.VMEM_SHARED((NROW, 128), F32),
          pltpu.VMEM((K,), I32),
          pltpu.VMEM((K,), I32),
          pltpu.VMEM((K,), I32),
          pltpu.VMEM((K, 128), F32),
          pltpu.VMEM((CSZ,), I32),
          pltpu.SemaphoreType.DMA,
      ],
  )
  def pool_k(x_hbm, labs_hbm, row2_hbm, sums_hbm,
             accs, rraw, gidx, sidx, xbuf, oidx, sem):
    c = lax.axis_index("c")
    s = lax.axis_index("s")
    cv = _cvec(c)

    def fill_row(e, carry):
      for g in range(8):
        xbuf[e, pl.ds(g * L, L)] = jnp.zeros((L,), F32)
      return carry
    lax.fori_loop(0, K, fill_row, 0)
    for q in range(RPT // CSZ):
      pltpu.sync_copy(xbuf.at[pl.ds(0, CSZ)],
                      accs.at[pl.ds(s * RPT + q * CSZ, CSZ)])
    plsc.subcore_barrier()

    def chunk(j, carry):
      base = (s * CH + j) * K
      pltpu.sync_copy(row2_hbm.at[pl.ds(base, K)], rraw)
      pltpu.sync_copy(labs_hbm.at[pl.ds(base, K)], sidx)
      for g in range(K // L):
        gidx[pl.ds(g * L, L)] = rraw[pl.ds(g * L, L)] + cv
      pltpu.async_copy(x_hbm.at[gidx], xbuf, sem).wait()
      pltpu.sync_copy(xbuf, accs.at[sidx], add=True)
      return carry
    lax.fori_loop(0, CH, chunk, 0)
    plsc.subcore_barrier()

    for q in range(NCO):
      r0 = s * RPT + q * CSZ
      pltpu.sync_copy(accs.at[pl.ds(r0, CSZ)], xbuf.at[pl.ds(0, CSZ)])
      pltpu.sync_copy(row2_hbm.at[pl.ds(r0, CSZ)], oidx)
      for g in range(CSZ // L):
        oidx[pl.ds(g * L, L)] = oidx[pl.ds(g * L, L)] + cv
      pltpu.sync_copy(xbuf.at[pl.ds(0, CSZ)], sums_hbm.at[oidx])

  return pool_k


def _make_gather1_kernel(NP_, TROW, KK=128):
  CH = NP_ // (NS * KK)

  @functools.partial(
      pl.kernel,
      out_type=jax.ShapeDtypeStruct((2 * NP_, 128), F32),
      mesh=_sc_mesh(),
      scratch_types=[
          pltpu.VMEM((KK,), I32),
          pltpu.VMEM((KK,), I32),
          pltpu.VMEM((KK,), I32),
          pltpu.VMEM((KK, 128), F32),
          pltpu.SemaphoreType.DMA,
      ],
  )
  def g1_k(t_hbm, lab2_hbm, row2_hbm, out_hbm, rraw, gidx, oidx, buf, sem):
    c = lax.axis_index("c")
    s = lax.axis_index("s")
    cv = _cvec(c)

    def chunk(j, carry):
      base = (s * CH + j) * KK
      pltpu.sync_copy(lab2_hbm.at[pl.ds(base, KK)], rraw)
      pltpu.sync_copy(row2_hbm.at[pl.ds(base, KK)], oidx)
      for g in range(KK // L):
        gidx[pl.ds(g * L, L)] = rraw[pl.ds(g * L, L)] + cv
        oidx[pl.ds(g * L, L)] = oidx[pl.ds(g * L, L)] + cv
      pltpu.async_copy(t_hbm.at[gidx], buf, sem).wait()
      pltpu.sync_copy(buf, out_hbm.at[oidx])
      return carry
    lax.fori_loop(0, CH, chunk, 0)

  return g1_k


def _make_unpool2_kernel(NP_, TROW, KK=128):
  CH = NP_ // (NS * KK)

  @functools.partial(
      pl.kernel,
      out_type=(jax.ShapeDtypeStruct((2 * NP_, 128), F32),
                jax.ShapeDtypeStruct((2 * NP_, 128), F32)),
      mesh=_sc_mesh(),
      scratch_types=[
          pltpu.VMEM((KK,), I32),
          pltpu.VMEM((KK,), I32),
          pltpu.VMEM((KK,), I32),
          pltpu.VMEM((KK, 128), F32),
          pltpu.SemaphoreType.DMA,
      ],
  )
  def up_k(t5_hbm, u6_hbm, p6_hbm, lab2_hbm, row2_hbm, h5_hbm, a6_hbm,
           rraw, gidx, oidx, buf, sem):
    c = lax.axis_index("c")
    s = lax.axis_index("s")
    cv = _cvec(c)

    def chunk(j, carry):
      base = (s * CH + j) * KK
      pltpu.sync_copy(lab2_hbm.at[pl.ds(base, KK)], rraw)
      pltpu.sync_copy(row2_hbm.at[pl.ds(base, KK)], oidx)
      for g in range(KK // L):
        gidx[pl.ds(g * L, L)] = rraw[pl.ds(g * L, L)] + cv
        oidx[pl.ds(g * L, L)] = oidx[pl.ds(g * L, L)] + cv
      pltpu.async_copy(t5_hbm.at[gidx], buf, sem).wait()
      pltpu.sync_copy(buf, h5_hbm.at[oidx])
      pltpu.async_copy(u6_hbm.at[gidx], buf, sem).wait()
      pltpu.async_copy(p6_hbm.at[oidx], buf, sem, add=True).wait()
      pltpu.sync_copy(buf, a6_hbm.at[oidx])
      return carry
    lax.fori_loop(0, CH, chunk, 0)

  return up_k


def _rows_spec(br, d):
  return pl.BlockSpec((br, d), lambda i: (i, 0))


def _full_spec(shape):
  return pl.BlockSpec(shape, lambda i: tuple(0 for _ in shape))


def _tc1_body(feat, pts, g1, w1a, w1b, b1, wm2a, wm2b, bm2, wm6b,
              h1_o, a2_o, b2n_o, p6_o, b6n_o):
  x1 = (jnp.dot(feat[...], w1a[...], preferred_element_type=F32)
        + jnp.dot(pts[...], w1b[...], preferred_element_type=F32) + b1[...])
  h1 = jnp.maximum(x1 - g1[...], 0.0)
  p2 = jnp.dot(pts[...], wm2b[...], preferred_element_type=F32)
  a2 = jnp.dot(h1, wm2a[...], preferred_element_type=F32) + p2 + bm2[...]
  p6 = jnp.dot(pts[...], wm6b[...], preferred_element_type=F32)
  h1_o[...] = h1
  a2_o[...] = a2
  b2n_o[...] = -p2
  p6_o[...] = p6
  b6n_o[...] = -p6


def _tc2_body(agg, wu, bu, hres, out):
  out[...] = jnp.maximum(
      jnp.dot(agg[...], wu[...], preferred_element_type=F32) + bu[...], 0.0
  ) + hres[...]


def _tc3_body(sums, cnt, cc, w3, b3, wm4a, wm4b, bm4,
              h3_o, a4_o, b4n_o):
  inv = 1.0 / jnp.maximum(cnt[...][:, 0:1], 1.0)
  pooled = sums[...] * inv
  h3 = jnp.maximum(jnp.dot(pooled, w3[...], preferred_element_type=F32)
                   + b3[...], 0.0)
  q4 = jnp.dot(cc[...], wm4b[...], preferred_element_type=F32)
  a4 = jnp.dot(h3, wm4a[...], preferred_element_type=F32) + q4 + bm4[...]
  h3_o[...] = h3
  a4_o[...] = a4
  b4n_o[...] = -q4


def _tc4_body(agg4, wu4, bu4, h3, w5, b5, wm6a, bm6, t5_o, u6_o):
  h4 = jnp.maximum(jnp.dot(agg4[...], wu4[...], preferred_element_type=F32)
                   + bu4[...], 0.0) + h3[...]
  t5 = jnp.maximum(jnp.dot(h4, w5[...], preferred_element_type=F32)
                   + b5[...], 0.0)
  u6 = jnp.dot(t5, wm6a[...], preferred_element_type=F32) + bm6[...]
  t5_o[...] = t5
  u6_o[...] = u6


def _tc5_body(agg6, wu6, bu6, h5, h2, wc, bc, out):
  h6 = jnp.maximum(jnp.dot(agg6[...], wu6[...], preferred_element_type=F32)
                   + bu6[...], 0.0) + h5[...]
  out[...] = jnp.dot(h6 + h2[...], wc[...],
                     preferred_element_type=F32) + bc[...]


def _tc0_body(cc, w1b, q1_o):
  q1_o[...] = jnp.dot(cc[...], w1b[...], preferred_element_type=F32)


def _hist_body(nbins, lab_ref, out_ref):
  i = pl.program_id(0)

  @pl.when(i == 0)
  def _():
    out_ref[...] = jnp.zeros_like(out_ref)

  lab = lab_ref[0, 0, :]
  nl = lab.shape[0]
  bins = lax.broadcasted_iota(I32, (nbins, nl), 0)
  oh = (bins == lab[None, :]).astype(F32)
  cnt = jnp.sum(oh, axis=1)
  out_ref[...] += jnp.broadcast_to(cnt[:, None], (nbins, 128))


def kernel(features, points, cluster_centers, l0_edges, l1_edges, labels,
           W1, b1, Wm2, bm2, Wu2, bu2, W3, b3, Wm4, bm4, Wu4, bu4,
           W5, b5, Wm6, bm6, Wu6, bu6, Wc, bc):
  n, d = features.shape
  m = cluster_centers.shape[0]
  e0 = l0_edges.shape[1]
  e1 = l1_edges.shape[1]
  c_out = Wc.shape[1]

  npad = ((n + NS * K - 1) // (NS * K)) * (NS * K)
  mpad = ((m + NS * 64 - 1) // (NS * 64)) * (NS * 64)
  KE0, KE1 = 112, 128
  ek0 = NS * KE0 * NBUF
  ek1 = NS * KE1 * NBUF
  e0pad = ((e0 + ek0 - 1) // ek0) * ek0
  e1pad = ((e1 + ek1 - 1) // ek1) * ek1
  DUMN = n + 2
  DUMM = m + 2

  f32 = functools.partial(jnp.asarray, dtype=F32)
  featp = jnp.pad(f32(features), ((0, npad - n), (0, 0)))
  ptsp = jnp.pad(f32(points), ((0, npad - n), (0, 0)))
  ccp = jnp.pad(f32(cluster_centers), ((0, mpad - m), (0, 0)))

  lab = labels.astype(I32)
  lab2 = jnp.pad(2 * lab, (0, npad - n))
  labS = jnp.pad(lab, (0, npad - n), constant_values=DUMM)
  row2n = (2 * jnp.arange(npad, dtype=I32))
  row2m = (2 * jnp.arange(mpad, dtype=I32))

  def pack_idx(edges, ne, epad, dum, kk):
    src = edges[0].astype(I32)
    dst = edges[1].astype(I32)
    src2 = jnp.pad(2 * src, (0, epad - ne))
    dst2 = jnp.pad(2 * dst, (0, epad - ne))
    dsts = jnp.pad(dst, (0, epad - ne), constant_values=dum)
    return jnp.stack([src2.reshape(-1, kk), dst2.reshape(-1, kk),
                      dsts.reshape(-1, kk)], axis=1).reshape(-1)

  idx0 = pack_idx(l0_edges, e0, e0pad, DUMN, KE0)
  idx1 = pack_idx(l1_edges, e1, e1pad, DUMM, KE1)

  r1 = lambda v: f32(v).reshape(1, -1)
  W1a, W1b = f32(W1[:d]), f32(W1[d:])
  Wm2a, Wm2b = f32(Wm2[:d]), f32(Wm2[d:])
  Wm4a, Wm4b = f32(Wm4[:d]), f32(Wm4[d:])
  Wm6a, Wm6b = f32(Wm6[:d]), f32(Wm6[d:])

  BRN = 1024
  GN = npad // BRN

  q1 = pl.pallas_call(
      _tc0_body,
      grid=(1,),
      in_specs=[_rows_spec(mpad, 3), _full_spec((3, d))],
      out_specs=_rows_spec(mpad, d),
      out_shape=jax.ShapeDtypeStruct((mpad, d), F32),
  )(ccp, W1b)

  g1_2n = _make_gather1_kernel(npad, 2 * mpad)(
      q1.reshape(2 * mpad, 128), lab2, row2n)
  g1 = g1_2n.reshape(npad, d)

  h1, a2, b2n, p6, b6n = pl.pallas_call(
      _tc1_body,
      grid=(GN,),
      in_specs=[_rows_spec(BRN, d), _rows_spec(BRN, 3), _rows_spec(BRN, d),
                _full_spec((d, d)), _full_spec((3, d)), _full_spec((1, d)),
                _full_spec((d, d)), _full_spec((3, d)), _full_spec((1, d)),
                _full_spec((3, d))],
      out_specs=[_rows_spec(BRN, d)] * 5,
      out_shape=[jax.ShapeDtypeStruct((npad, d), F32)] * 5,
  )(featp, ptsp, g1, W1a, W1b, r1(b1), Wm2a, Wm2b, r1(bm2), Wm6b)

  edge_n = _make_edge_kernel(e0pad, npad, 2 * npad, KE=KE0)

  agg2_2n = edge_n(a2.reshape(2 * npad, 128), b2n.reshape(2 * npad, 128),
                   idx0, row2n)
  agg2 = agg2_2n.reshape(npad, d)

  h2 = pl.pallas_call(
      _tc2_body,
      grid=(GN,),
      in_specs=[_rows_spec(BRN, d), _full_spec((d, d)), _full_spec((1, d)),
                _rows_spec(BRN, d)],
      out_specs=_rows_spec(BRN, d),
      out_shape=jax.ShapeDtypeStruct((npad, d), F32),
  )(agg2, f32(Wu2), r1(bu2), h1)

  sums_2n = _make_pool_kernel(npad, mpad)(
      h2.reshape(2 * npad, 128), labS, row2n)
  sums = sums_2n.reshape(mpad, d)

  LBLK = 1024
  cnt = pl.pallas_call(
      functools.partial(_hist_body, mpad),
      grid=(npad // LBLK,),
      in_specs=[pl.BlockSpec((1, 1, LBLK), lambda i: (i, 0, 0))],
      out_specs=_full_spec((mpad, 128)),
      out_shape=jax.ShapeDtypeStruct((mpad, 128), F32),
  )(labS.reshape(npad // LBLK, 1, LBLK))

  h3, a4, b4n = pl.pallas_call(
      _tc3_body,
      grid=(1,),
      in_specs=[_rows_spec(mpad, d), _rows_spec(mpad, 128), _rows_spec(mpad, 3),
                _full_spec((d, d)), _full_spec((1, d)),
                _full_spec((d, d)), _full_spec((3, d)), _full_spec((1, d))],
      out_specs=[_rows_spec(mpad, d)] * 3,
      out_shape=[jax.ShapeDtypeStruct((mpad, d), F32)] * 3,
  )(sums, cnt, ccp, f32(W3), r1(b3), Wm4a, Wm4b, r1(bm4))

  agg4_2n = _make_edge_kernel(e1pad, mpad, 2 * mpad, KE=KE1)(
      a4.reshape(2 * mpad, 128), b4n.reshape(2 * mpad, 128),
      idx1, row2m)
  agg4 = agg4_2n.reshape(mpad, d)

  t5, u6 = pl.pallas_call(
      _tc4_body,
      grid=(1,),
      in_specs=[_rows_spec(mpad, d), _full_spec((d, d)), _full_spec((1, d)),
                _rows_spec(mpad, d), _full_spec((d, d)), _full_spec((1, d)),
                _full_spec((d, d)), _full_spec((1, d))],
      out_specs=[_rows_spec(mpad, d)] * 2,
      out_shape=[jax.ShapeDtypeStruct((mpad, d), F32)] * 2,
  )(agg4, f32(Wu4), r1(bu4), h3, f32(W5), r1(b5), Wm6a, r1(bm6))

  h5_2n, a6_2n = _make_unpool2_kernel(npad, 2 * mpad)(
      t5.reshape(2 * mpad, 128), u6.reshape(2 * mpad, 128),
      p6.reshape(2 * npad, 128), lab2, row2n)
  h5 = h5_2n.reshape(npad, d)

  agg6_2n = edge_n(a6_2n, b6n.reshape(2 * npad, 128),
                   idx0, row2n)
  agg6 = agg6_2n.reshape(npad, d)

  out = pl.pallas_call(
      _tc5_body,
      grid=(GN,),
      in_specs=[_rows_spec(BRN, d), _full_spec((d, d)), _full_spec((1, d)),
                _rows_spec(BRN, d), _rows_spec(BRN, d),
                _full_spec((d, c_out)), _full_spec((1, c_out))],
      out_specs=_rows_spec(BRN, c_out),
      out_shape=jax.ShapeDtypeStruct((npad, c_out), F32),
  )(agg6, f32(Wu6), r1(bu6), h5, h2, f32(Wc), r1(bc))

  return out[:n]


# speedup vs baseline: 1.1295x; 1.0537x over previous
import functools

import jax
import jax.numpy as jnp
from jax import lax
from jax.experimental import pallas as pl
from jax.experimental.pallas import tpu as pltpu
from jax.experimental.pallas import tpu_sc as plsc

NC = 2
NS = 16
L = 16
K = 128

F32 = jnp.float32
I32 = jnp.int32


def _sc_mesh():
  return plsc.VectorSubcoreMesh(
      core_axis_name="c", subcore_axis_name="s", num_cores=NC, num_subcores=NS)


def _cvec(c):
  return jnp.broadcast_to(c, (L,)).astype(I32)


NBUF = 2


def _csz(rpt, ke):
  c = (min(rpt, ke, 128) // 16) * 16
  while rpt % c:
    c -= 16
  return c


def _make_edge_kernel(EPAD, NROW, TROW, NBUF=NBUF, KE=K):
  CH = EPAD // (NS * KE)
  assert CH % NBUF == 0
  RPT = NROW // NS
  CSZ = _csz(RPT, KE)
  NCO = RPT // CSZ

  @functools.partial(
      pl.kernel,
      out_type=jax.ShapeDtypeStruct((2 * NROW, 128), F32),
      mesh=_sc_mesh(),
      scratch_types=[
          pltpu.VMEM_SHARED((NROW, 128), F32),
          [pltpu.VMEM((3 * KE,), I32)] * NBUF,
          [pltpu.VMEM((KE,), I32)] * NBUF,
          [pltpu.VMEM((KE,), I32)] * NBUF,
          [pltpu.VMEM((KE,), I32)] * NBUF,
          [pltpu.VMEM((KE, 128), F32)] * NBUF,
          pltpu.VMEM((KE, 128), F32),
          pltpu.VMEM((CSZ,), I32),
          [pltpu.SemaphoreType.DMA] * NBUF,
          [pltpu.SemaphoreType.DMA] * NBUF,
          [pltpu.SemaphoreType.DMA] * NBUF,
      ],
  )
  def edge_k(a_hbm, bn_hbm, idx_hbm, row2_hbm, out_hbm,
             acc, idxb, gsrc, gdst, sidx, abuf, bbuf, oidx,
             sema, semb, sems):
    c = lax.axis_index("c")
    s = lax.axis_index("s")
    cv = _cvec(c)

    def zero_row(e, carry):
      for g in range(8):
        abuf[0][e, pl.ds(g * L, L)] = jnp.zeros((L,), F32)
      return carry
    lax.fori_loop(0, KE, zero_row, 0)
    for q in range(RPT // CSZ):
      pltpu.sync_copy(abuf[0].at[pl.ds(0, CSZ)],
                      acc.at[pl.ds(s * RPT + q * CSZ, CSZ)])
    plsc.subcore_barrier()

    def group(jj, carry):
      for b in range(NBUF):
        @pl.when(jj > 0)
        def _():
          pltpu.make_async_copy(abuf[b], acc.at[sidx[b]], sems[b]).wait()
        base = ((s * CH + jj * NBUF + b) * 3) * KE
        pltpu.sync_copy(idx_hbm.at[pl.ds(base, 3 * KE)], idxb[b])
        for g in range(KE // L):
          gsrc[b][pl.ds(g * L, L)] = idxb[b][pl.ds(g * L, L)] + cv
          gdst[b][pl.ds(g * L, L)] = idxb[b][pl.ds(KE + g * L, L)] + cv
          sidx[b][pl.ds(g * L, L)] = idxb[b][pl.ds(2 * KE + g * L, L)]
        pltpu.async_copy(a_hbm.at[gsrc[b]], abuf[b], sema[b])
        if b == 0:
          pltpu.async_copy(bn_hbm.at[gdst[0]], bbuf, semb[0])
      for b in range(NBUF):
        pltpu.make_async_copy(a_hbm.at[gsrc[b]], abuf[b], sema[b]).wait()
        pltpu.make_async_copy(bn_hbm.at[gdst[b]], bbuf, semb[b]).wait()
        a_ = abuf[b]

        def msg_row(e, inner):
          for g in range(8):
            va = a_[e, pl.ds(g * L, L)]
            vb = bbuf[e, pl.ds(g * L, L)]
            a_[e, pl.ds(g * L, L)] = jnp.maximum(va + vb, 0.0)
          return inner
        lax.fori_loop(0, KE, msg_row, 0)
        if b + 1 < NBUF:
          pltpu.async_copy(bn_hbm.at[gdst[b + 1]], bbuf, semb[b + 1])
        pltpu.async_copy(abuf[b], acc.at[sidx[b]], sems[b], add=True)
      return carry
    lax.fori_loop(0, CH // NBUF, group, 0)
    for b in range(NBUF):
      pltpu.make_async_copy(abuf[b], acc.at[sidx[b]], sems[b]).wait()
    plsc.subcore_barrier()

    for q in range(NCO):
      r0 = s * RPT + q * CSZ
      pltpu.sync_copy(acc.at[pl.ds(r0, CSZ)], abuf[0].at[pl.ds(0, CSZ)])
      pltpu.sync_copy(row2_hbm.at[pl.ds(r0, CSZ)], oidx)
      for g in range(CSZ // L):
        oidx[pl.ds(g * L, L)] = oidx[pl.ds(g * L, L)] + cv
      pltpu.sync_copy(abuf[0].at[pl.ds(0, CSZ)], out_hbm.at[oidx])

  return edge_k


def _make_pool_kernel(NP_, NROW):
  CH = NP_ // (NS * K)
  RPT = NROW // NS
  CSZ = min(128, RPT)
  NCO = RPT // CSZ

  @functools.partial(
      pl.kernel,
      out_type=jax.ShapeDtypeStruct((2 * NROW, 128), F32),
      mesh=_sc_mesh(),
      scratch_types=[
          pltpu.VMEM_SHARED((NROW, 128), F32),
          pltpu.VMEM((K,), I32),
          pltpu.VMEM((K,), I32),
          pltpu.VMEM((K,), I32),
          pltpu.VMEM((K, 128), F32),
          pltpu.VMEM((CSZ,), I32),
          pltpu.SemaphoreType.DMA,
      ],
  )
  def pool_k(x_hbm, labs_hbm, row2_hbm, sums_hbm,
             accs, rraw, gidx, sidx, xbuf, oidx, sem):
    c = lax.axis_index("c")
    s = lax.axis_index("s")
    cv = _cvec(c)

    def fill_row(e, carry):
      for g in range(8):
        xbuf[e, pl.ds(g * L, L)] = jnp.zeros((L,), F32)
      return carry
    lax.fori_loop(0, K, fill_row, 0)
    for q in range(RPT // CSZ):
      pltpu.sync_copy(xbuf.at[pl.ds(0, CSZ)],
                      accs.at[pl.ds(s * RPT + q * CSZ, CSZ)])
    plsc.subcore_barrier()

    def chunk(j, carry):
      base = (s * CH + j) * K
      pltpu.sync_copy(row2_hbm.at[pl.ds(base, K)], rraw)
      pltpu.sync_copy(labs_hbm.at[pl.ds(base, K)], sidx)
      for g in range(K // L):
        gidx[pl.ds(g * L, L)] = rraw[pl.ds(g * L, L)] + cv
      pltpu.async_copy(x_hbm.at[gidx], xbuf, sem).wait()
      pltpu.sync_copy(xbuf, accs.at[sidx], add=True)
      return carry
    lax.fori_loop(0, CH, chunk, 0)
    plsc.subcore_barrier()

    for q in range(NCO):
      r0 = s * RPT + q * CSZ
      pltpu.sync_copy(accs.at[pl.ds(r0, CSZ)], xbuf.at[pl.ds(0, CSZ)])
      pltpu.sync_copy(row2_hbm.at[pl.ds(r0, CSZ)], oidx)
      for g in range(CSZ // L):
        oidx[pl.ds(g * L, L)] = oidx[pl.ds(g * L, L)] + cv
      pltpu.sync_copy(xbuf.at[pl.ds(0, CSZ)], sums_hbm.at[oidx])

  return pool_k


def _make_gather1_kernel(NP_, TROW, KK=128):
  CH = NP_ // (NS * KK)

  @functools.partial(
      pl.kernel,
      out_type=jax.ShapeDtypeStruct((2 * NP_, 128), F32),
      mesh=_sc_mesh(),
      scratch_types=[
          pltpu.VMEM((KK,), I32),
          pltpu.VMEM((KK,), I32),
          pltpu.VMEM((KK,), I32),
          pltpu.VMEM((KK, 128), F32),
          pltpu.SemaphoreType.DMA,
      ],
  )
  def g1_k(t_hbm, lab2_hbm, row2_hbm, out_hbm, rraw, gidx, oidx, buf, sem):
    c = lax.axis_index("c")
    s = lax.axis_index("s")
    cv = _cvec(c)

    def chunk(j, carry):
      base = (s * CH + j) * KK
      pltpu.sync_copy(lab2_hbm.at[pl.ds(base, KK)], rraw)
      pltpu.sync_copy(row2_hbm.at[pl.ds(base, KK)], oidx)
      for g in range(KK // L):
        gidx[pl.ds(g * L, L)] = rraw[pl.ds(g * L, L)] + cv
        oidx[pl.ds(g * L, L)] = oidx[pl.ds(g * L, L)] + cv
      pltpu.async_copy(t_hbm.at[gidx], buf, sem).wait()
      pltpu.sync_copy(buf, out_hbm.at[oidx])
      return carry
    lax.fori_loop(0, CH, chunk, 0)

  return g1_k


def _make_unpool2_kernel(NP_, TROW, KK=128):
  CH = NP_ // (NS * KK)

  @functools.partial(
      pl.kernel,
      out_type=(jax.ShapeDtypeStruct((2 * NP_, 128), F32),
                jax.ShapeDtypeStruct((2 * NP_, 128), F32)),
      mesh=_sc_mesh(),
      scratch_types=[
          pltpu.VMEM((KK,), I32),
          pltpu.VMEM((KK,), I32),
          pltpu.VMEM((KK,), I32),
          pltpu.VMEM((KK, 128), F32),
          pltpu.SemaphoreType.DMA,
      ],
  )
  def up_k(t5_hbm, u6_hbm, p6_hbm, lab2_hbm, row2_hbm, h5_hbm, a6_hbm,
           rraw, gidx, oidx, buf, sem):
    c = lax.axis_index("c")
    s = lax.axis_index("s")
    cv = _cvec(c)

    def chunk(j, carry):
      base = (s * CH + j) * KK
      pltpu.sync_copy(lab2_hbm.at[pl.ds(base, KK)], rraw)
      pltpu.sync_copy(row2_hbm.at[pl.ds(base, KK)], oidx)
      for g in range(KK // L):
        gidx[pl.ds(g * L, L)] = rraw[pl.ds(g * L, L)] + cv
        oidx[pl.ds(g * L, L)] = oidx[pl.ds(g * L, L)] + cv
      pltpu.async_copy(t5_hbm.at[gidx], buf, sem).wait()
      pltpu.sync_copy(buf, h5_hbm.at[oidx])
      pltpu.async_copy(u6_hbm.at[gidx], buf, sem).wait()
      pltpu.async_copy(p6_hbm.at[oidx], buf, sem, add=True).wait()
      pltpu.sync_copy(buf, a6_hbm.at[oidx])
      return carry
    lax.fori_loop(0, CH, chunk, 0)

  return up_k


def _rows_spec(br, d):
  return pl.BlockSpec((br, d), lambda i: (i, 0))


def _full_spec(shape):
  return pl.BlockSpec(shape, lambda i: tuple(0 for _ in shape))


def _tc1_body(feat, pts, g1, w1a, w1b, b1, wm2a, wm2b, bm2, wm6b,
              h1_o, a2_o, b2n_o, p6_o, b6n_o):
  x1 = (jnp.dot(feat[...], w1a[...], preferred_element_type=F32)
        + jnp.dot(pts[...], w1b[...], preferred_element_type=F32) + b1[...])
  h1 = jnp.maximum(x1 - g1[...], 0.0)
  p2 = jnp.dot(pts[...], wm2b[...], preferred_element_type=F32)
  a2 = jnp.dot(h1, wm2a[...], preferred_element_type=F32) + p2 + bm2[...]
  p6 = jnp.dot(pts[...], wm6b[...], preferred_element_type=F32)
  h1_o[...] = h1
  a2_o[...] = a2
  b2n_o[...] = -p2
  p6_o[...] = p6
  b6n_o[...] = -p6


def _tc2_body(agg, wu, bu, hres, out):
  out[...] = jnp.maximum(
      jnp.dot(agg[...], wu[...], preferred_element_type=F32) + bu[...], 0.0
  ) + hres[...]


def _tc3_body(sums, cnt, cc, w3, b3, wm4a, wm4b, bm4,
              h3_o, a4_o, b4n_o):
  inv = 1.0 / jnp.maximum(cnt[...][:, 0:1], 1.0)
  pooled = sums[...] * inv
  h3 = jnp.maximum(jnp.dot(pooled, w3[...], preferred_element_type=F32)
                   + b3[...], 0.0)
  q4 = jnp.dot(cc[...], wm4b[...], preferred_element_type=F32)
  a4 = jnp.dot(h3, wm4a[...], preferred_element_type=F32) + q4 + bm4[...]
  h3_o[...] = h3
  a4_o[...] = a4
  b4n_o[...] = -q4


def _tc4_body(agg4, wu4, bu4, h3, w5, b5, wm6a, bm6, t5_o, u6_o):
  h4 = jnp.maximum(jnp.dot(agg4[...], wu4[...], preferred_element_type=F32)
                   + bu4[...], 0.0) + h3[...]
  t5 = jnp.maximum(jnp.dot(h4, w5[...], preferred_element_type=F32)
                   + b5[...], 0.0)
  u6 = jnp.dot(t5, wm6a[...], preferred_element_type=F32) + bm6[...]
  t5_o[...] = t5
  u6_o[...] = u6


def _tc5_body(agg6, wu6, bu6, h5, h2, wc, bc, out):
  h6 = jnp.maximum(jnp.dot(agg6[...], wu6[...], preferred_element_type=F32)
                   + bu6[...], 0.0) + h5[...]
  out[...] = jnp.dot(h6 + h2[...], wc[...],
                     preferred_element_type=F32) + bc[...]


def _tc0_body(cc, w1b, q1_o):
  q1_o[...] = jnp.dot(cc[...], w1b[...], preferred_element_type=F32)


def _hist_body(nbins, lab_ref, out_ref):
  i = pl.program_id(0)

  @pl.when(i == 0)
  def _():
    out_ref[...] = jnp.zeros_like(out_ref)

  lab = lab_ref[0, 0, :]
  nl = lab.shape[0]
  bins = lax.broadcasted_iota(I32, (nbins, nl), 0)
  oh = (bins == lab[None, :]).astype(F32)
  cnt = jnp.sum(oh, axis=1)
  out_ref[...] += jnp.broadcast_to(cnt[:, None], (nbins, 128))


def kernel(features, points, cluster_centers, l0_edges, l1_edges, labels,
           W1, b1, Wm2, bm2, Wu2, bu2, W3, b3, Wm4, bm4, Wu4, bu4,
           W5, b5, Wm6, bm6, Wu6, bu6, Wc, bc):
  n, d = features.shape
  m = cluster_centers.shape[0]
  e0 = l0_edges.shape[1]
  e1 = l1_edges.shape[1]
  c_out = Wc.shape[1]

  npad = ((n + NS * K - 1) // (NS * K)) * (NS * K)
  mpad = ((m + NS * 64 - 1) // (NS * 64)) * (NS * 64)
  KE0, KE1 = 112, 128
  ek0 = NS * KE0 * NBUF
  ek1 = NS * KE1 * NBUF
  e0pad = ((e0 + ek0 - 1) // ek0) * ek0
  e1pad = ((e1 + ek1 - 1) // ek1) * ek1
  DUMN = n + 2
  DUMM = m + 2

  f32 = functools.partial(jnp.asarray, dtype=F32)
  featp = jnp.pad(f32(features), ((0, npad - n), (0, 0)))
  ptsp = jnp.pad(f32(points), ((0, npad - n), (0, 0)))
  ccp = jnp.pad(f32(cluster_centers), ((0, mpad - m), (0, 0)))

  lab = labels.astype(I32)
  lab2 = jnp.pad(2 * lab, (0, npad - n))
  labS = jnp.pad(lab, (0, npad - n), constant_values=DUMM)
  row2n = (2 * jnp.arange(npad, dtype=I32))
  row2m = (2 * jnp.arange(mpad, dtype=I32))

  def pack_idx(edges, ne, epad, dum, kk):
    src = edges[0].astype(I32)
    dst = edges[1].astype(I32)
    src2 = jnp.pad(2 * src, (0, epad - ne))
    dst2 = jnp.pad(2 * dst, (0, epad - ne))
    dsts = jnp.pad(dst, (0, epad - ne), constant_values=dum)
    return jnp.stack([src2.reshape(-1, kk), dst2.reshape(-1, kk),
                      dsts.reshape(-1, kk)], axis=1).reshape(-1)

  idx0 = pack_idx(l0_edges, e0, e0pad, DUMN, KE0)
  idx1 = pack_idx(l1_edges, e1, e1pad, DUMM, KE1)

  r1 = lambda v: f32(v).reshape(1, -1)
  W1a, W1b = f32(W1[:d]), f32(W1[d:])
  Wm2a, Wm2b = f32(Wm2[:d]), f32(Wm2[d:])
  Wm4a, Wm4b = f32(Wm4[:d]), f32(Wm4[d:])
  Wm6a, Wm6b = f32(Wm6[:d]), f32(Wm6[d:])

  BRN = 1024
  GN = npad // BRN

  q1 = pl.pallas_call(
      _tc0_body,
      grid=(1,),
      in_specs=[_rows_spec(mpad, 3), _full_spec((3, d))],
      out_specs=_rows_spec(mpad, d),
      out_shape=jax.ShapeDtypeStruct((mpad, d), F32),
  )(ccp, W1b)

  g1_2n = _make_gather1_kernel(npad, 2 * mpad)(
      q1.reshape(2 * mpad, 128), lab2, row2n)
  g1 = g1_2n.reshape(npad, d)

  h1, a2, b2n, p6, b6n = pl.pallas_call(
      _tc1_body,
      grid=(GN,),
      in_specs=[_rows_spec(BRN, d), _rows_spec(BRN, 3), _rows_spec(BRN, d),
                _full_spec((d, d)), _full_spec((3, d)), _full_spec((1, d)),
                _full_spec((d, d)), _full_spec((3, d)), _full_spec((1, d)),
                _full_spec((3, d))],
      out_specs=[_rows_spec(BRN, d)] * 5,
      out_shape=[jax.ShapeDtypeStruct((npad, d), F32)] * 5,
  )(featp, ptsp, g1, W1a, W1b, r1(b1), Wm2a, Wm2b, r1(bm2), Wm6b)

  edge_n = _make_edge_kernel(e0pad, npad, 2 * npad, KE=KE0)

  agg2_2n = edge_n(a2.reshape(2 * npad, 128), b2n.reshape(2 * npad, 128),
                   idx0, row2n)
  agg2 = agg2_2n.reshape(npad, d)

  h2 = pl.pallas_call(
      _tc2_body,
      grid=(GN,),
      in_specs=[_rows_spec(BRN, d), _full_spec((d, d)), _full_spec((1, d)),
                _rows_spec(BRN, d)],
      out_specs=_rows_spec(BRN, d),
      out_shape=jax.ShapeDtypeStruct((npad, d), F32),
  )(agg2, f32(Wu2), r1(bu2), h1)

  sums_2n = _make_pool_kernel(npad, mpad)(
      h2.reshape(2 * npad, 128), labS, row2n)
  sums = sums_2n.reshape(mpad, d)

  LBLK = 1024
  cnt = pl.pallas_call(
      functools.partial(_hist_body, mpad),
      grid=(npad // LBLK,),
      in_specs=[pl.BlockSpec((1, 1, LBLK), lambda i: (i, 0, 0))],
      out_specs=_full_spec((mpad, 128)),
      out_shape=jax.ShapeDtypeStruct((mpad, 128), F32),
  )(labS.reshape(npad // LBLK, 1, LBLK))

  h3, a4, b4n = pl.pallas_call(
      _tc3_body,
      grid=(1,),
      in_specs=[_rows_spec(mpad, d), _rows_spec(mpad, 128), _rows_spec(mpad, 3),
                _full_spec((d, d)), _full_spec((1, d)),
                _full_spec((d, d)), _full_spec((3, d)), _full_spec((1, d))],
      out_specs=[_rows_spec(mpad, d)] * 3,
      out_shape=[jax.ShapeDtypeStruct((mpad, d), F32)] * 3,
  )(sums, cnt, ccp, f32(W3), r1(b3), Wm4a, Wm4b, r1(bm4))

  agg4_2n = _make_edge_kernel(e1pad, mpad, 2 * mpad, KE=KE1)(
      a4.reshape(2 * mpad, 128), b4n.reshape(2 * mpad, 128),
      idx1, row2m)
  agg4 = agg4_2n.reshape(mpad, d)

  t5, u6 = pl.pallas_call(
      _tc4_body,
      grid=(1,),
      in_specs=[_rows_spec(mpad, d), _full_spec((d, d)), _full_spec((1, d)),
                _rows_spec(mpad, d), _full_spec((d, d)), _full_spec((1, d)),
                _full_spec((d, d)), _full_spec((1, d))],
      out_specs=[_rows_spec(mpad, d)] * 2,
      out_shape=[jax.ShapeDtypeStruct((mpad, d), F32)] * 2,
  )(agg4, f32(Wu4), r1(bu4), h3, f32(W5), r1(b5), Wm6a, r1(bm6))

  h5_2n, a6_2n = _make_unpool2_kernel(npad, 2 * mpad)(
      t5.reshape(2 * mpad, 128), u6.reshape(2 * mpad, 128),
      p6.reshape(2 * npad, 128), lab2, row2n)
  h5 = h5_2n.reshape(npad, d)

  agg6_2n = edge_n(a6_2n, b6n.reshape(2 * npad, 128),
                   idx0, row2n)
  agg6 = agg6_2n.reshape(npad, d)

  out = pl.pallas_call(
      _tc5_body,
      grid=(GN,),
      in_specs=[_rows_spec(BRN, d), _full_spec((d, d)), _full_spec((1, d)),
                _rows_spec(BRN, d), _rows_spec(BRN, d),
                _full_spec((d, c_out)), _full_spec((1, c_out))],
      out_specs=_rows_spec(BRN, c_out),
      out_shape=jax.ShapeDtypeStruct((npad, c_out), F32),
  )(agg6, f32(Wu6), r1(bu6), h5, h2, f32(Wc), r1(bc))

  return out[:n]
